# baseline (speedup 1.0000x reference)
import functools

import jax
import jax.numpy as jnp
from jax import lax
from jax.experimental import pallas as pl
from jax.experimental.pallas import tpu as pltpu
from jax.experimental.pallas import tpu_sc as plsc

EMBEDDING_DIM = 64
R = 256
G = 2


@functools.lru_cache(maxsize=None)
def _build(B, D, NC, NS):
    NW = NC * NS
    b_per_w = B // NW
    S = b_per_w // R
    T = S // (2 * G)

    mesh = plsc.VectorSubcoreMesh(core_axis_name="c", subcore_axis_name="s")

    @functools.partial(
        pl.kernel,
        mesh=mesh,
        out_type=jax.ShapeDtypeStruct((B, D), jnp.float32),
        scratch_types=[
            pltpu.VMEM((S, R), jnp.int32),
            pltpu.VMEM((G * R, D), jnp.float32),
            pltpu.VMEM((G * R, D), jnp.float32),
            pltpu.SemaphoreType.DMA,
            pltpu.SemaphoreType.DMA,
            pltpu.SemaphoreType.DMA,
            pltpu.SemaphoreType.DMA,
        ],
        compiler_params=pltpu.CompilerParams(use_tc_tiling_on_sc=False),
    )
    def gather_kernel(table_hbm, idx_hbm, out_hbm,
                      idx_v, rows_a, rows_b, gs_a, gs_b, os_a, os_b):
        wid = lax.axis_index("s") * NC + lax.axis_index("c")
        base = wid * b_per_w
        pltpu.sync_copy(idx_hbm.at[wid], idx_v)

        def fire_gathers(g, rows, sem):
            for b in range(G):
                s = jnp.minimum(g * G + b, S - 1)
                pltpu.async_copy(table_hbm.at[idx_v.at[s]],
                                 rows.at[pl.ds(b * R, R)], sem)

        def wait_gathers(g, rows, sem):
            for b in range(G):
                s = jnp.minimum(g * G + b, S - 1)
                pltpu.make_async_copy(table_hbm.at[idx_v.at[s]],
                                      rows.at[pl.ds(b * R, R)], sem).wait()

        def fire_outs(g, rows, sem):
            pltpu.async_copy(rows, out_hbm.at[pl.ds(base + g * G * R, G * R)], sem)

        def wait_outs(g, rows, sem):
            pltpu.make_async_copy(rows, out_hbm.at[pl.ds(base + g * G * R, G * R)], sem).wait()

        fire_gathers(0, rows_a, gs_a)
        fire_gathers(1, rows_b, gs_b)

        def body(t, carry):
            g0 = 2 * t
            wait_gathers(g0, rows_a, gs_a)
            fire_outs(g0, rows_a, os_a)
            wait_gathers(g0 + 1, rows_b, gs_b)
            fire_outs(g0 + 1, rows_b, os_b)
            wait_outs(g0, rows_a, os_a)
            fire_gathers(g0 + 2, rows_a, gs_a)
            wait_outs(g0 + 1, rows_b, os_b)
            fire_gathers(g0 + 3, rows_b, gs_b)
            return carry

        lax.fori_loop(0, T, body, 0)
        wait_gathers(2 * T, rows_a, gs_a)
        wait_gathers(2 * T + 1, rows_b, gs_b)

    return gather_kernel


def kernel(token_ids, weight):
    B = token_ids.shape[0] * token_ids.shape[1]
    D = weight.shape[1]
    info = plsc.get_sparse_core_info()
    NC, NS = info.num_cores, info.num_subcores
    idx = token_ids.reshape(-1).astype(jnp.int32)
    idx3 = idx.reshape(NC * NS, B // (NC * NS) // R, R)
    out = _build(B, D, NC, NS)(weight, idx3)
    return out.reshape(token_ids.shape[0], token_ids.shape[1], D)


# speedup vs baseline: 1.0034x; 1.0034x over previous
import functools

import jax
import jax.numpy as jnp
from jax import lax
from jax.experimental import pallas as pl
from jax.experimental.pallas import tpu as pltpu
from jax.experimental.pallas import tpu_sc as plsc

EMBEDDING_DIM = 64
R = 128
G = 2
NBANKS = 4


@functools.lru_cache(maxsize=None)
def _build(B, D, NC, NS):
    NW = NC * NS
    b_per_w = B // NW
    S = b_per_w // R
    ngroups = S // G
    T = ngroups // NBANKS

    mesh = plsc.VectorSubcoreMesh(core_axis_name="c", subcore_axis_name="s")

    @functools.partial(
        pl.kernel,
        mesh=mesh,
        out_type=jax.ShapeDtypeStruct((B, D), jnp.float32),
        scratch_types=(
            [pltpu.VMEM((S, R), jnp.int32)]
            + [pltpu.VMEM((G * R, D), jnp.float32)] * NBANKS
            + [pltpu.SemaphoreType.DMA] * (2 * NBANKS)
        ),
        compiler_params=pltpu.CompilerParams(use_tc_tiling_on_sc=False),
    )
    def gather_kernel(table_hbm, idx_hbm, out_hbm, idx_v, *bufs):
        rows = bufs[:NBANKS]
        gsems = bufs[NBANKS:2 * NBANKS]
        osems = bufs[2 * NBANKS:]
        wid = lax.axis_index("s") * NC + lax.axis_index("c")
        base = wid * b_per_w
        pltpu.sync_copy(idx_hbm.at[wid], idx_v)

        def fire_gathers(g, k):
            for b in range(G):
                s = jnp.minimum(g * G + b, S - 1)
                pltpu.async_copy(table_hbm.at[idx_v.at[s]],
                                 rows[k].at[pl.ds(b * R, R)], gsems[k])

        def wait_gathers(g, k):
            for b in range(G):
                s = jnp.minimum(g * G + b, S - 1)
                pltpu.make_async_copy(table_hbm.at[idx_v.at[s]],
                                      rows[k].at[pl.ds(b * R, R)], gsems[k]).wait()

        def fire_outs(g, k):
            pltpu.async_copy(rows[k], out_hbm.at[pl.ds(base + g * G * R, G * R)],
                             osems[k])

        def wait_outs(g, k):
            pltpu.make_async_copy(rows[k], out_hbm.at[pl.ds(base + g * G * R, G * R)],
                                  osems[k]).wait()

        for k in range(NBANKS):
            fire_gathers(k, k)

        def body(t, carry):
            g0 = NBANKS * t
            for k in range(NBANKS):
                wait_gathers(g0 + k, k)
                fire_outs(g0 + k, k)
            for k in range(NBANKS):
                wait_outs(g0 + k, k)
                fire_gathers(g0 + NBANKS + k, k)
            return carry

        lax.fori_loop(0, T, body, 0)
        for k in range(NBANKS):
            wait_gathers(NBANKS * T + k, k)

    return gather_kernel


def kernel(token_ids, weight):
    B = token_ids.shape[0] * token_ids.shape[1]
    D = weight.shape[1]
    info = plsc.get_sparse_core_info()
    NC, NS = info.num_cores, info.num_subcores
    idx = token_ids.reshape(-1).astype(jnp.int32)
    idx3 = idx.reshape(NC * NS, B // (NC * NS) // R, R)
    out = _build(B, D, NC, NS)(weight, idx3)
    return out.reshape(token_ids.shape[0], token_ids.shape[1], D)
